# Initial kernel scaffold; baseline (speedup 1.0000x reference)
import jax
import jax.numpy as jnp
from jax.experimental import pallas as pl


def kernel(inputs, W, b, P0):
    raise NotImplementedError("write your pallas kernel here")


# speedup vs baseline: 1.8268x; 1.8268x over previous
import jax
import jax.numpy as jnp
from jax.experimental import pallas as pl
from jax.experimental.pallas import tpu as pltpu

_WORKER = 1000
_TASK = 20000
_ET = 2
_AB = 128
_K = _TASK * _ET
_WB = 8


def _body(b_ref, wf_ref, w_ref, tau_ref, out_ref):
    z = jnp.dot(wf_ref[...], w_ref[...],
                preferred_element_type=jnp.float32) + b_ref[0]
    lp2 = -(jnp.maximum(z, 0.0) + jnp.log1p(jnp.exp(-jnp.abs(z))))
    out_ref[...] = jnp.exp(lp2 + tau_ref[...] * z)


def kernel(inputs, W, b, P0):
    wf = inputs[:_WORKER]
    tau = inputs[_WORKER:, :_ET].reshape(1, _K)
    out = pl.pallas_call(
        _body,
        grid=(_WORKER // _WB,),
        in_specs=[
            pl.BlockSpec(memory_space=pltpu.SMEM),
            pl.BlockSpec((_WB, _AB), lambda i: (i, 0)),
            pl.BlockSpec((_AB, 1), lambda i: (0, 0)),
            pl.BlockSpec((1, _K), lambda i: (0, 0)),
        ],
        out_specs=pl.BlockSpec((_WB, _K), lambda i: (i, 0)),
        out_shape=jax.ShapeDtypeStruct((_WORKER, _K), jnp.float32),
    )(b, wf, W, tau)
    return out.reshape(_WORKER, _TASK, _ET)


# speedup vs baseline: 15.9026x; 8.7052x over previous
import jax
import jax.numpy as jnp
from jax.experimental import pallas as pl
from jax.experimental.pallas import tpu as pltpu

_WORKER = 1000
_TASK = 20000
_ET = 2
_AB = 128
_K = _TASK * _ET
_WB = 8


def _body(b_ref, wf_ref, w_ref, tau_ref, out_ref):
    z = jnp.dot(wf_ref[...], w_ref[...],
                preferred_element_type=jnp.float32) + b_ref[0]
    lp2 = -(jnp.maximum(z, 0.0) + jnp.log1p(jnp.exp(-jnp.abs(z))))
    out_ref[...] = jnp.exp(lp2[:, :, None] + tau_ref[...] * z[:, :, None])


def kernel(inputs, W, b, P0):
    wf = inputs[:_WORKER]
    tau = inputs[_WORKER:, :_ET].T[None]
    out = pl.pallas_call(
        _body,
        grid=(_WORKER // _WB,),
        in_specs=[
            pl.BlockSpec(memory_space=pltpu.SMEM),
            pl.BlockSpec((_WB, _AB), lambda i: (i, 0)),
            pl.BlockSpec((_AB, 1), lambda i: (0, 0)),
            pl.BlockSpec((1, _ET, _TASK), lambda i: (0, 0, 0)),
        ],
        out_specs=pl.BlockSpec((_WB, _ET, _TASK), lambda i: (i, 0, 0)),
        out_shape=jax.ShapeDtypeStruct((_WORKER, _ET, _TASK), jnp.float32),
    )(b, wf, W, tau)
    return out.transpose(0, 2, 1)


# speedup vs baseline: 27.2159x; 1.7114x over previous
import jax
import jax.numpy as jnp
from jax.experimental import pallas as pl
from jax.experimental.pallas import tpu as pltpu

_WORKER = 1000
_TASK = 20000
_ET = 2
_AB = 128
_K = _TASK * _ET
_WB = 40


def _body(b_ref, wf_ref, w_ref, tau_ref, out_ref):
    z = jnp.dot(wf_ref[...], w_ref[...],
                preferred_element_type=jnp.float32) + b_ref[0]
    lp2 = -(jnp.maximum(z, 0.0) + jnp.log1p(jnp.exp(-jnp.abs(z))))
    out_ref[...] = jnp.exp(lp2[:, :, None] + tau_ref[...] * z[:, :, None])


def kernel(inputs, W, b, P0):
    wf = inputs[:_WORKER]
    tau = inputs[_WORKER:, :_ET].T[None]
    out = pl.pallas_call(
        _body,
        grid=(_WORKER // _WB,),
        in_specs=[
            pl.BlockSpec(memory_space=pltpu.SMEM),
            pl.BlockSpec((_WB, _AB), lambda i: (i, 0)),
            pl.BlockSpec((_AB, 1), lambda i: (0, 0)),
            pl.BlockSpec((1, _ET, _TASK), lambda i: (0, 0, 0)),
        ],
        out_specs=pl.BlockSpec((_WB, _ET, _TASK), lambda i: (i, 0, 0)),
        out_shape=jax.ShapeDtypeStruct((_WORKER, _ET, _TASK), jnp.float32),
    )(b, wf, W, tau)
    return out.transpose(0, 2, 1)


# speedup vs baseline: 27.3390x; 1.0045x over previous
import jax
import jax.numpy as jnp
from jax.experimental import pallas as pl
from jax.experimental.pallas import tpu as pltpu

_WORKER = 1000
_TASK = 20000
_ET = 2
_AB = 128
_K = _TASK * _ET
_WB = 80


def _body(b_ref, wf_ref, w_ref, tau_ref, out_ref):
    z = jnp.dot(wf_ref[...], w_ref[...],
                preferred_element_type=jnp.float32) + b_ref[0]
    lp2 = -(jnp.maximum(z, 0.0) + jnp.log1p(jnp.exp(-jnp.abs(z))))
    out_ref[...] = jnp.exp(lp2[:, :, None] + tau_ref[...] * z[:, :, None])


def kernel(inputs, W, b, P0):
    wf = inputs[:_WORKER]
    tau = inputs[_WORKER:, :_ET].T[None]
    out = pl.pallas_call(
        _body,
        grid=(_WORKER // _WB,),
        in_specs=[
            pl.BlockSpec(memory_space=pltpu.SMEM),
            pl.BlockSpec((_WB, _AB), lambda i: (i, 0)),
            pl.BlockSpec((_AB, 1), lambda i: (0, 0)),
            pl.BlockSpec((1, _ET, _TASK), lambda i: (0, 0, 0)),
        ],
        out_specs=pl.BlockSpec((_WB, _ET, _TASK), lambda i: (i, 0, 0)),
        out_shape=jax.ShapeDtypeStruct((_WORKER, _ET, _TASK), jnp.float32),
    )(b, wf, W, tau)
    return out.transpose(0, 2, 1)
